# Initial kernel scaffold; baseline (speedup 1.0000x reference)
import functools

import jax
import jax.numpy as jnp
from jax import lax
from jax.experimental import pallas as pl
from jax.experimental.pallas import tpu as pltpu
from jax.experimental.pallas import tpu_sc as plsc

N = 10000
E = 320000
D = 128

NC_SC = 2
NS = 16
NW = NC_SC * NS
CH = 128
NCHUNK = 80
EPT = CH * NCHUNK
EPAD = EPT * NW
N_PAD = 10240
RPT = N_PAD // NS
RING = 4
DEGW = 16

_mesh = plsc.VectorSubcoreMesh(core_axis_name="c", subcore_axis_name="s")


def _zero_buf(buf, nrows, ncols):
    @pl.loop(0, nrows)
    def _(r):
        @pl.loop(0, ncols, step=16)
        def _(cc):
            buf[r, pl.ds(cc, 16)] = jnp.zeros((16,), jnp.float32)


def _fill_ones(buf, nrows):
    @pl.loop(0, nrows)
    def _(r):
        buf[r, pl.ds(0, 16)] = jnp.ones((16,), jnp.float32)


@functools.partial(
    pl.kernel,
    out_type=jax.ShapeDtypeStruct((NC_SC, N_PAD, DEGW), jnp.float32),
    mesh=_mesh,
    scratch_types=[
        pltpu.VMEM_SHARED((N_PAD, DEGW), jnp.float32),
        pltpu.VMEM((NCHUNK, CH), jnp.int32),
        pltpu.VMEM((CH, DEGW), jnp.float32),
        pltpu.SemaphoreType.DMA,
        pltpu.SemaphoreType.DMA,
        pltpu.SemaphoreType.DMA,
        pltpu.SemaphoreType.DMA,
    ],
)
def _sc_deg(dst_hbm, out_hbm, deg_acc, dst_v, ones_v, s0, s1, s2, s3):
    ssem = (s0, s1, s2, s3)
    c = lax.axis_index("c")
    s = lax.axis_index("s")
    wid = c * NS + s

    _zero_buf(ones_v, CH, DEGW)
    for k in range(RPT // CH):
        pltpu.sync_copy(ones_v,
                        deg_acc.at[pl.ds(s * RPT + k * CH, CH)])
    _fill_ones(ones_v, CH)
    pltpu.sync_copy(dst_hbm.at[pl.ds(wid * NCHUNK, NCHUNK)], dst_v)
    plsc.subcore_barrier()

    @pl.loop(0, NCHUNK, step=RING)
    def _(g0):
        for b in range(RING):
            i = g0 + b

            @pl.when(i >= RING)
            def _():
                pltpu.make_async_copy(ones_v, deg_acc.at[dst_v.at[i]],
                                      ssem[b]).wait()

            pltpu.async_copy(ones_v, deg_acc.at[dst_v.at[i]], ssem[b],
                             add=True)

    for b in range(RING):
        pltpu.make_async_copy(ones_v, deg_acc.at[dst_v.at[0]], ssem[b]).wait()
    plsc.subcore_barrier()
    pltpu.sync_copy(deg_acc.at[pl.ds(s * RPT, RPT)],
                    out_hbm.at[c].at[pl.ds(s * RPT, RPT)])


@functools.partial(
    pl.kernel,
    out_type=jax.ShapeDtypeStruct((NC_SC, N_PAD, D), jnp.float32),
    mesh=_mesh,
    scratch_types=[
        pltpu.VMEM_SHARED((N_PAD, D), jnp.float32),
        pltpu.VMEM((NCHUNK, CH), jnp.int32),
        pltpu.VMEM((NCHUNK, CH), jnp.int32),
        pltpu.VMEM((CH, D), jnp.float32),
        pltpu.VMEM((CH, D), jnp.float32),
        pltpu.VMEM((CH, D), jnp.float32),
        pltpu.VMEM((CH, D), jnp.float32),
        pltpu.SemaphoreType.DMA,
        pltpu.SemaphoreType.DMA,
        pltpu.SemaphoreType.DMA,
        pltpu.SemaphoreType.DMA,
        pltpu.SemaphoreType.DMA,
        pltpu.SemaphoreType.DMA,
        pltpu.SemaphoreType.DMA,
        pltpu.SemaphoreType.DMA,
    ],
)
def _sc_msg(s_hbm, src_hbm, dst_hbm, out_hbm, acc, src_v, dst_v,
            r0, r1, r2, r3, g0, g1, g2, g3, t0, t1, t2, t3):
    rows = (r0, r1, r2, r3)
    gsem = (g0, g1, g2, g3)
    ssem = (t0, t1, t2, t3)
    c = lax.axis_index("c")
    s = lax.axis_index("s")
    wid = c * NS + s

    _zero_buf(r0, CH, D)
    for k in range(RPT // CH):
        pltpu.sync_copy(r0, acc.at[pl.ds(s * RPT + k * CH, CH)])

    pltpu.sync_copy(src_hbm.at[pl.ds(wid * NCHUNK, NCHUNK)], src_v)
    pltpu.sync_copy(dst_hbm.at[pl.ds(wid * NCHUNK, NCHUNK)], dst_v)

    for b in range(RING):
        pltpu.async_copy(s_hbm.at[src_v.at[b]], rows[b], gsem[b])
    plsc.subcore_barrier()

    @pl.loop(0, NCHUNK, step=RING)
    def _(gbase):
        for b in range(RING):
            i = gbase + b
            pltpu.make_async_copy(s_hbm.at[src_v.at[i]], rows[b],
                                  gsem[b]).wait()
            pltpu.async_copy(rows[b], acc.at[dst_v.at[i]], ssem[b], add=True)
            pltpu.make_async_copy(rows[b], acc.at[dst_v.at[i]],
                                  ssem[b]).wait()
            ni = i + RING

            @pl.when(ni < NCHUNK)
            def _():
                pltpu.async_copy(s_hbm.at[src_v.at[ni]], rows[b], gsem[b])

    plsc.subcore_barrier()
    pltpu.sync_copy(acc.at[pl.ds(s * RPT, RPT)],
                    out_hbm.at[c].at[pl.ds(s * RPT, RPT)])


_RB = 1000
_GRID = N // _RB


def _mm_body(x_ref, w_ref, o_ref):
    o_ref[...] = jnp.dot(x_ref[...], w_ref[...],
                         preferred_element_type=jnp.float32)


def _tc_matmul(x, w):
    return pl.pallas_call(
        _mm_body,
        grid=(_GRID,),
        in_specs=[
            pl.BlockSpec((_RB, D), lambda i: (i, 0)),
            pl.BlockSpec((D, D), lambda i: (0, 0)),
        ],
        out_specs=pl.BlockSpec((_RB, D), lambda i: (i, 0)),
        out_shape=jax.ShapeDtypeStruct((N, D), jnp.float32),
    )(x, w)


def _scale_body(d0_ref, d1_ref, hw_ref, s_ref, dn_ref):
    deg = d0_ref[:, :1] + d1_ref[:, :1] + 1.0
    dn = lax.rsqrt(deg)
    dn_ref[...] = dn
    s_ref[...] = hw_ref[...] * dn


def _tc_scale(d0, d1, hw):
    return pl.pallas_call(
        _scale_body,
        grid=(_GRID,),
        in_specs=[
            pl.BlockSpec((_RB, DEGW), lambda i: (i, 0)),
            pl.BlockSpec((_RB, DEGW), lambda i: (i, 0)),
            pl.BlockSpec((_RB, D), lambda i: (i, 0)),
        ],
        out_specs=[
            pl.BlockSpec((_RB, D), lambda i: (i, 0)),
            pl.BlockSpec((_RB, 1), lambda i: (i, 0)),
        ],
        out_shape=[
            jax.ShapeDtypeStruct((N, D), jnp.float32),
            jax.ShapeDtypeStruct((N, 1), jnp.float32),
        ],
    )(d0, d1, hw)


def _layer_body(a0_ref, a1_ref, sp_ref, dn_ref, b_ref, w_ref, o_ref):
    dn = dn_ref[...]
    t = (a0_ref[...] + a1_ref[...] + sp_ref[...]) * dn + b_ref[...]
    h = jnp.maximum(t, 0.0)
    o_ref[...] = jnp.dot(h, w_ref[...],
                         preferred_element_type=jnp.float32) * dn


def _tc_layer(a0, a1, sp, dn, bias, w):
    return pl.pallas_call(
        _layer_body,
        grid=(_GRID,),
        in_specs=[
            pl.BlockSpec((_RB, D), lambda i: (i, 0)),
            pl.BlockSpec((_RB, D), lambda i: (i, 0)),
            pl.BlockSpec((_RB, D), lambda i: (i, 0)),
            pl.BlockSpec((_RB, 1), lambda i: (i, 0)),
            pl.BlockSpec((1, D), lambda i: (0, 0)),
            pl.BlockSpec((D, D), lambda i: (0, 0)),
        ],
        out_specs=pl.BlockSpec((_RB, D), lambda i: (i, 0)),
        out_shape=jax.ShapeDtypeStruct((N, D), jnp.float32),
    )(a0, a1, sp, dn, bias, w)


def _final_body(a0_ref, a1_ref, sp_ref, dn_ref, b_ref, o_ref):
    t = (a0_ref[...] + a1_ref[...] + sp_ref[...]) * dn_ref[...] + b_ref[...]
    o_ref[...] = jnp.maximum(t, 0.0)


def _tc_final(a0, a1, sp, dn, bias):
    return pl.pallas_call(
        _final_body,
        grid=(_GRID,),
        in_specs=[
            pl.BlockSpec((_RB, D), lambda i: (i, 0)),
            pl.BlockSpec((_RB, D), lambda i: (i, 0)),
            pl.BlockSpec((_RB, D), lambda i: (i, 0)),
            pl.BlockSpec((_RB, 1), lambda i: (i, 0)),
            pl.BlockSpec((1, D), lambda i: (0, 0)),
        ],
        out_specs=pl.BlockSpec((_RB, D), lambda i: (i, 0)),
        out_shape=jax.ShapeDtypeStruct((N, D), jnp.float32),
    )(a0, a1, sp, dn, bias)


def kernel(x, g, W0, b0, W1, b1, W2, b2):
    pad = EPAD - E
    srcp = jnp.concatenate([g[0], jnp.zeros((pad,), jnp.int32)])
    dstp = jnp.concatenate([g[1], jnp.full((pad,), N, jnp.int32)])
    srcp = srcp.reshape(EPAD // CH, CH)
    dstp = dstp.reshape(EPAD // CH, CH)

    degp = _sc_deg(dstp)
    hw0 = _tc_matmul(x, W0)
    s0, dn = _tc_scale(degp[0, :N], degp[1, :N], hw0)

    acc = _sc_msg(s0, srcp, dstp)
    s1 = _tc_layer(acc[0, :N], acc[1, :N], s0, dn, b0.reshape(1, D), W1)
    acc = _sc_msg(s1, srcp, dstp)
    s2 = _tc_layer(acc[0, :N], acc[1, :N], s1, dn, b1.reshape(1, D), W2)
    acc = _sc_msg(s2, srcp, dstp)
    return _tc_final(acc[0, :N], acc[1, :N], s2, dn, b2.reshape(1, D))


# speedup vs baseline: 7.9792x; 7.9792x over previous
import functools

import jax
import jax.numpy as jnp
from jax import lax
from jax.experimental import pallas as pl
from jax.experimental.pallas import tpu as pltpu
from jax.experimental.pallas import tpu_sc as plsc

N = 10000
E = 320000
D = 128

NC_SC = 2
NS = 16
NW = NC_SC * NS
CH = 128
NCHUNK = 80
EPT = CH * NCHUNK
EPAD = EPT * NW
N_PAD = 10240
RPT = N_PAD // NS
RING = 2
DRING = 4
DEGW = 16

_mesh = plsc.VectorSubcoreMesh(core_axis_name="c", subcore_axis_name="s")


def _zero_buf(buf, nrows, ncols):
    @pl.loop(0, nrows)
    def _(r):
        @pl.loop(0, ncols, step=16)
        def _(cc):
            buf[r, pl.ds(cc, 16)] = jnp.zeros((16,), jnp.float32)


def _fill_ones(buf, nrows):
    @pl.loop(0, nrows)
    def _(r):
        buf[r, pl.ds(0, 16)] = jnp.ones((16,), jnp.float32)


@functools.partial(
    pl.kernel,
    out_type=jax.ShapeDtypeStruct((NC_SC, N_PAD, DEGW), jnp.float32),
    mesh=_mesh,
    scratch_types=[
        pltpu.VMEM_SHARED((N_PAD, DEGW), jnp.float32),
        pltpu.VMEM((NCHUNK, CH), jnp.int32),
        pltpu.VMEM((CH, DEGW), jnp.float32),
        pltpu.SemaphoreType.DMA,
        pltpu.SemaphoreType.DMA,
        pltpu.SemaphoreType.DMA,
        pltpu.SemaphoreType.DMA,
    ],
)
def _sc_deg(dst_hbm, out_hbm, deg_acc, dst_v, ones_v, s0, s1, s2, s3):
    ssem = (s0, s1, s2, s3)
    c = lax.axis_index("c")
    s = lax.axis_index("s")
    wid = c * NS + s

    _zero_buf(ones_v, CH, DEGW)
    for k in range(RPT // CH):
        pltpu.sync_copy(ones_v,
                        deg_acc.at[pl.ds(s * RPT + k * CH, CH)])
    _fill_ones(ones_v, CH)
    pltpu.sync_copy(dst_hbm.at[pl.ds(wid * NCHUNK, NCHUNK)], dst_v)
    plsc.subcore_barrier()

    @pl.loop(0, NCHUNK, step=DRING)
    def _(g0):
        for b in range(DRING):
            i = g0 + b

            @pl.when(i >= DRING)
            def _():
                pltpu.make_async_copy(ones_v, deg_acc.at[dst_v.at[i]],
                                      ssem[b]).wait()

            pltpu.async_copy(ones_v, deg_acc.at[dst_v.at[i]], ssem[b],
                             add=True)

    for b in range(DRING):
        pltpu.make_async_copy(ones_v, deg_acc.at[dst_v.at[0]], ssem[b]).wait()
    plsc.subcore_barrier()
    pltpu.sync_copy(deg_acc.at[pl.ds(s * RPT, RPT)],
                    out_hbm.at[c].at[pl.ds(s * RPT, RPT)])


IRING = 4


@functools.partial(
    pl.kernel,
    out_type=jax.ShapeDtypeStruct((NC_SC, N_PAD, D), jnp.float32),
    mesh=_mesh,
    scratch_types=[
        pltpu.VMEM_SHARED((N_PAD, D), jnp.float32),
        pltpu.VMEM((1, 2, CH), jnp.int32),
        pltpu.VMEM((1, 2, CH), jnp.int32),
        pltpu.VMEM((1, 2, CH), jnp.int32),
        pltpu.VMEM((1, 2, CH), jnp.int32),
        pltpu.VMEM((CH, D), jnp.float32),
        pltpu.VMEM((CH, D), jnp.float32),
        pltpu.SemaphoreType.DMA,
        pltpu.SemaphoreType.DMA,
        pltpu.SemaphoreType.DMA,
        pltpu.SemaphoreType.DMA,
        pltpu.SemaphoreType.DMA,
        pltpu.SemaphoreType.DMA,
        pltpu.SemaphoreType.DMA,
        pltpu.SemaphoreType.DMA,
    ],
)
def _sc_msg(s_hbm, gidx_hbm, out_hbm, acc, ix0, ix1, ix2, ix3, r0, r1,
            i0, i1, i2, i3, g0, g1, t0, t1):
    ix = (ix0, ix1, ix2, ix3)
    rows = (r0, r1)
    isem = (i0, i1, i2, i3)
    gsem = (g0, g1)
    ssem = (t0, t1)
    c = lax.axis_index("c")
    s = lax.axis_index("s")
    wid = c * NS + s
    base = wid * NCHUNK

    _zero_buf(r0, CH, D)
    for k in range(RPT // CH):
        pltpu.sync_copy(r0, acc.at[pl.ds(s * RPT + k * CH, CH)])

    for b in range(IRING):
        pltpu.async_copy(gidx_hbm.at[pl.ds(base + b, 1)], ix[b], isem[b])
    for b in range(RING):
        pltpu.make_async_copy(gidx_hbm.at[pl.ds(base + b, 1)], ix[b],
                              isem[b]).wait()
        pltpu.async_copy(s_hbm.at[ix[b].at[0, 0]], rows[b], gsem[b])
    plsc.subcore_barrier()

    @pl.loop(0, NCHUNK, step=IRING)
    def _(gbase):
        for b in range(IRING):
            i = gbase + b
            b2 = b % RING
            pltpu.make_async_copy(s_hbm.at[ix[b].at[0, 0]], rows[b2],
                                  gsem[b2]).wait()
            pltpu.async_copy(rows[b2], acc.at[ix[b].at[0, 1]], ssem[b2],
                             add=True)
            pltpu.make_async_copy(rows[b2], acc.at[ix[b].at[0, 1]],
                                  ssem[b2]).wait()
            ni4 = i + IRING

            @pl.when(ni4 < NCHUNK)
            def _():
                pltpu.async_copy(gidx_hbm.at[pl.ds(base + ni4, 1)], ix[b],
                                 isem[b])

            ni2 = i + RING
            bn = (b + RING) % IRING

            @pl.when(ni2 < NCHUNK)
            def _():
                pltpu.make_async_copy(gidx_hbm.at[pl.ds(base + ni2, 1)],
                                      ix[bn], isem[bn]).wait()
                pltpu.async_copy(s_hbm.at[ix[bn].at[0, 0]], rows[b2],
                                 gsem[b2])

    plsc.subcore_barrier()
    pltpu.sync_copy(acc.at[pl.ds(s * RPT, RPT)],
                    out_hbm.at[c].at[pl.ds(s * RPT, RPT)])


_RB = 1000
_GRID = N // _RB


def _mm_body(x_ref, w_ref, o_ref):
    o_ref[...] = jnp.dot(x_ref[...], w_ref[...],
                         preferred_element_type=jnp.float32)


def _tc_matmul(x, w):
    return pl.pallas_call(
        _mm_body,
        grid=(_GRID,),
        in_specs=[
            pl.BlockSpec((_RB, D), lambda i: (i, 0)),
            pl.BlockSpec((D, D), lambda i: (0, 0)),
        ],
        out_specs=pl.BlockSpec((_RB, D), lambda i: (i, 0)),
        out_shape=jax.ShapeDtypeStruct((N, D), jnp.float32),
    )(x, w)


def _scale_body(d0_ref, d1_ref, hw_ref, s_ref, dn_ref):
    deg = d0_ref[:, :1] + d1_ref[:, :1] + 1.0
    dn = lax.rsqrt(deg)
    dn_ref[...] = dn
    s_ref[...] = hw_ref[...] * dn


def _tc_scale(d0, d1, hw):
    return pl.pallas_call(
        _scale_body,
        grid=(_GRID,),
        in_specs=[
            pl.BlockSpec((_RB, DEGW), lambda i: (i, 0)),
            pl.BlockSpec((_RB, DEGW), lambda i: (i, 0)),
            pl.BlockSpec((_RB, D), lambda i: (i, 0)),
        ],
        out_specs=[
            pl.BlockSpec((_RB, D), lambda i: (i, 0)),
            pl.BlockSpec((_RB, 1), lambda i: (i, 0)),
        ],
        out_shape=[
            jax.ShapeDtypeStruct((N, D), jnp.float32),
            jax.ShapeDtypeStruct((N, 1), jnp.float32),
        ],
    )(d0, d1, hw)


def _layer_body(a0_ref, a1_ref, sp_ref, dn_ref, b_ref, w_ref, o_ref):
    dn = dn_ref[...]
    t = (a0_ref[...] + a1_ref[...] + sp_ref[...]) * dn + b_ref[...]
    h = jnp.maximum(t, 0.0)
    o_ref[...] = jnp.dot(h, w_ref[...],
                         preferred_element_type=jnp.float32) * dn


def _tc_layer(a0, a1, sp, dn, bias, w):
    return pl.pallas_call(
        _layer_body,
        grid=(_GRID,),
        in_specs=[
            pl.BlockSpec((_RB, D), lambda i: (i, 0)),
            pl.BlockSpec((_RB, D), lambda i: (i, 0)),
            pl.BlockSpec((_RB, D), lambda i: (i, 0)),
            pl.BlockSpec((_RB, 1), lambda i: (i, 0)),
            pl.BlockSpec((1, D), lambda i: (0, 0)),
            pl.BlockSpec((D, D), lambda i: (0, 0)),
        ],
        out_specs=pl.BlockSpec((_RB, D), lambda i: (i, 0)),
        out_shape=jax.ShapeDtypeStruct((N, D), jnp.float32),
    )(a0, a1, sp, dn, bias, w)


def _final_body(a0_ref, a1_ref, sp_ref, dn_ref, b_ref, o_ref):
    t = (a0_ref[...] + a1_ref[...] + sp_ref[...]) * dn_ref[...] + b_ref[...]
    o_ref[...] = jnp.maximum(t, 0.0)


def _tc_final(a0, a1, sp, dn, bias):
    return pl.pallas_call(
        _final_body,
        grid=(_GRID,),
        in_specs=[
            pl.BlockSpec((_RB, D), lambda i: (i, 0)),
            pl.BlockSpec((_RB, D), lambda i: (i, 0)),
            pl.BlockSpec((_RB, D), lambda i: (i, 0)),
            pl.BlockSpec((_RB, 1), lambda i: (i, 0)),
            pl.BlockSpec((1, D), lambda i: (0, 0)),
        ],
        out_specs=pl.BlockSpec((_RB, D), lambda i: (i, 0)),
        out_shape=jax.ShapeDtypeStruct((N, D), jnp.float32),
    )(a0, a1, sp, dn, bias)


def kernel(x, g, W0, b0, W1, b1, W2, b2):
    pad = EPAD - E
    srcp = jnp.concatenate([g[0], jnp.zeros((pad,), jnp.int32)])
    dstp = jnp.concatenate([g[1], jnp.full((pad,), N, jnp.int32)])
    srcp = srcp.reshape(EPAD // CH, CH)
    dstp = dstp.reshape(EPAD // CH, CH)
    gidx = jnp.stack([srcp, dstp], axis=1)

    degp = _sc_deg(dstp)
    hw0 = _tc_matmul(x, W0)
    s0, dn = _tc_scale(degp[0, :N], degp[1, :N], hw0)

    acc = _sc_msg(s0, gidx)
    s1 = _tc_layer(acc[0, :N], acc[1, :N], s0, dn, b0.reshape(1, D), W1)
    acc = _sc_msg(s1, gidx)
    s2 = _tc_layer(acc[0, :N], acc[1, :N], s1, dn, b1.reshape(1, D), W2)
    acc = _sc_msg(s2, gidx)
    return _tc_final(acc[0, :N], acc[1, :N], s2, dn, b2.reshape(1, D))
